# Initial kernel scaffold; baseline (speedup 1.0000x reference)
import jax
import jax.numpy as jnp
from jax.experimental import pallas as pl


def kernel(coors, feats, emb_w, emb_b, ew1, eb1, ew2, eb2, nnw, nnb, nw1, nb1, nw2, nb2, cw1, cb1, cw2, cb2, cns, ln_w, ln_b):
    raise NotImplementedError("write your pallas kernel here")


# speedup vs baseline: 1.0362x; 1.0362x over previous
import functools

import jax
import jax.numpy as jnp
from jax.experimental import pallas as pl

N = 10000
IN_DIM = 128
F = 64
M = 64
L = 3
EIN = 2 * F + 1
CUTOFF = 0.09
E_MAX = 700000

EDGE_BLK = 2000


def _silu(x):
    return x * jax.nn.sigmoid(x)


def _edge_mlp_body(xi_ref, xj_ref, rel_ref, vf_ref,
                   wi_ref, wj_ref, wr_ref, b1_ref,
                   w2_ref, b2_ref, cw1_ref, cb1_ref, cw2_ref, cb2_ref,
                   cns_ref,
                   m_ref, wrel_ref):
    xi = xi_ref[...]
    xj = xj_ref[...]
    rel = rel_ref[...]
    vf = vf_ref[...]
    rd = jnp.sum(rel * rel, axis=-1, keepdims=True)
    pre = (jnp.dot(xi, wi_ref[...], preferred_element_type=jnp.float32)
           + jnp.dot(xj, wj_ref[...], preferred_element_type=jnp.float32)
           + rd * wr_ref[...] + b1_ref[...])
    m1 = _silu(pre)
    m = _silu(jnp.dot(m1, w2_ref[...], preferred_element_type=jnp.float32)
              + b2_ref[...])
    t = _silu(jnp.dot(m, cw1_ref[...], preferred_element_type=jnp.float32)
              + cb1_ref[...])
    w = jnp.dot(t, cw2_ref[...], preferred_element_type=jnp.float32) + cb2_ref[...]
    nrm = jnp.sqrt(rd)
    rel_n = rel / jnp.maximum(nrm, 1e-8) * cns_ref[0, 0]
    m_ref[...] = m * vf
    wrel_ref[...] = w * rel_n * vf


@functools.partial(jax.jit, static_argnames=())
def _edge_mlp(xi, xj, rel, vf, wi, wj, wr, b1, w2, b2, cw1, cb1, cw2, cb2, cns):
    grid = (E_MAX // EDGE_BLK,)
    eb = lambda i: (i, 0)
    full = lambda i: (0, 0)
    return pl.pallas_call(
        _edge_mlp_body,
        grid=grid,
        in_specs=[
            pl.BlockSpec((EDGE_BLK, F), eb),
            pl.BlockSpec((EDGE_BLK, F), eb),
            pl.BlockSpec((EDGE_BLK, 3), eb),
            pl.BlockSpec((EDGE_BLK, 1), eb),
            pl.BlockSpec((F, 2 * EIN), full),
            pl.BlockSpec((F, 2 * EIN), full),
            pl.BlockSpec((1, 2 * EIN), full),
            pl.BlockSpec((1, 2 * EIN), full),
            pl.BlockSpec((2 * EIN, M), full),
            pl.BlockSpec((1, M), full),
            pl.BlockSpec((M, 4 * M), full),
            pl.BlockSpec((1, 4 * M), full),
            pl.BlockSpec((4 * M, 1), full),
            pl.BlockSpec((1, 1), full),
            pl.BlockSpec((1, 1), full),
        ],
        out_specs=[
            pl.BlockSpec((EDGE_BLK, M), eb),
            pl.BlockSpec((EDGE_BLK, 3), eb),
        ],
        out_shape=[
            jax.ShapeDtypeStruct((E_MAX, M), jnp.float32),
            jax.ShapeDtypeStruct((E_MAX, 3), jnp.float32),
        ],
    )(xi, xj, rel, vf, wi, wj, wr, b1, w2, b2, cw1, cb1, cw2, cb2, cns)


def kernel(coors, feats, emb_w, emb_b, ew1, eb1, ew2, eb2, nnw, nnb,
           nw1, nb1, nw2, nb2, cw1, cb1, cw2, cb2, cns, ln_w, ln_b):
    n = coors.shape[0]
    r2 = jnp.sum(coors * coors, axis=1)
    d2 = r2[:, None] + r2[None, :] - 2.0 * (coors @ coors.T)
    mask = (d2 <= CUTOFF * CUTOFF) & (~jnp.eye(n, dtype=bool))
    num = jnp.sum(mask)
    dst, src = jnp.nonzero(mask, size=E_MAX, fill_value=0)
    vf = (jnp.arange(E_MAX) < num).astype(jnp.float32)[:, None]

    h = feats @ emb_w + emb_b
    c = coors
    for l in range(L):
        xi = h[dst]
        xj = h[src]
        rel = c[src] - c[dst]
        m, wrel = _edge_mlp(
            xi, xj, rel, vf,
            ew1[l][:F], ew1[l][F:2 * F], ew1[l][2 * F:2 * F + 1],
            eb1[l][None, :], ew2[l], eb2[l][None, :],
            cw1[l], cb1[l][None, :], cw2[l], cb2[l][None, :],
            cns[l][None, :],
        )
        c = c + jnp.zeros_like(c).at[dst].add(wrel)
        m_i = jnp.zeros((n, M), dtype=h.dtype).at[dst].add(m)
        mu = jnp.mean(h)
        sd = jnp.std(h)
        hf = (h - mu) / (sd + 1e-5) * nnw[l] + nnb[l]
        ho = _silu(jnp.concatenate([hf, m_i], axis=-1) @ nw1[l] + nb1[l]) @ nw2[l] + nb2[l]
        z = 2.0 * h + ho
        zm = jnp.mean(z, axis=-1, keepdims=True)
        zv = jnp.var(z, axis=-1, keepdims=True)
        h = (z - zm) / jnp.sqrt(zv + 1e-5) * ln_w + ln_b
    return c, h
